# baseline (speedup 1.0000x reference)
import jax
import jax.numpy as jnp
from jax import lax
from jax.experimental import pallas as pl
from jax.experimental.pallas import tpu as pltpu
from jax.experimental.pallas import tpu_sc as plsc

B = 16384
C = 1000
NC = 2
NS = 16
NW = NC * NS
RPW = B // NW
R = 32
NCH = RPW // R
CHUNK = R * C
LANES = 16
ZUNROLL = 8


def _body(cond_hbm, out_hbm, idx_v, buf0, buf1, sem0, sem1):
    wid = lax.axis_index("s") * NC + lax.axis_index("c")
    base = wid * RPW
    pltpu.sync_copy(cond_hbm.at[pl.ds(base, RPW)], idx_v)

    zeros = jnp.zeros((LANES,), jnp.float32)
    ones = jnp.ones((LANES,), jnp.float32)

    def zbody(i, carry):
        for j in range(ZUNROLL):
            off = (i * ZUNROLL + j) * LANES
            buf0[pl.ds(off, LANES)] = zeros
            buf1[pl.ds(off, LANES)] = zeros
        return carry
    lax.fori_loop(0, CHUNK // (LANES * ZUNROLL), zbody, 0)

    rowmul = lax.iota(jnp.int32, LANES) * C
    bufs = (buf0, buf1)
    sems = (sem0, sem1)

    def chunk_pos(k, g):
        cond16 = idx_v[pl.ds(k * R + g * LANES, LANES)]
        return rowmul + (g * LANES * C) + cond16

    def dst(k):
        return out_hbm.at[pl.ds((base + k * R) * C, CHUNK)]

    for k in range(NCH):
        b = k % 2
        if k >= 2:
            pltpu.make_async_copy(bufs[b], dst(k - 2), sems[b]).wait()
            for g in range(R // LANES):
                plsc.store_scatter(bufs[b], [chunk_pos(k - 2, g)], zeros)
        for g in range(R // LANES):
            plsc.store_scatter(bufs[b], [chunk_pos(k, g)], ones)
        pltpu.async_copy(bufs[b], dst(k), sems[b])

    for k in (NCH - 2, NCH - 1):
        pltpu.make_async_copy(bufs[k % 2], dst(k), sems[k % 2]).wait()


def kernel(cond):
    mesh = plsc.VectorSubcoreMesh(
        core_axis_name="c", subcore_axis_name="s", num_cores=NC
    )
    out = pl.kernel(
        _body,
        out_type=jax.ShapeDtypeStruct((B * C,), jnp.float32),
        mesh=mesh,
        compiler_params=pltpu.CompilerParams(needs_layout_passes=False),
        scratch_types=[
            pltpu.VMEM((RPW,), jnp.int32),
            pltpu.VMEM((CHUNK,), jnp.float32),
            pltpu.VMEM((CHUNK,), jnp.float32),
            pltpu.SemaphoreType.DMA,
            pltpu.SemaphoreType.DMA,
        ],
    )(cond)
    return out.reshape(B, 1, C)
